# Initial kernel scaffold; baseline (speedup 1.0000x reference)
import jax
import jax.numpy as jnp
from jax.experimental import pallas as pl


def kernel(x, edge_index, edge_attr, batch, parity_atoms, add_feature, Wn, bn, We, be, Wc0, bc0, Wc1, bc1, Wc2, bc2, W1, b1, Wf0, bf0, Wf1, bf1, Wo, bo):
    raise NotImplementedError("write your pallas kernel here")


# speedup vs baseline: 6.7002x; 6.7002x over previous
import functools

import jax
import jax.numpy as jnp
from jax import lax
from jax.experimental import pallas as pl
from jax.experimental.pallas import tpu as pltpu
from jax.experimental.pallas import tpu_sc as plsc

N = 10000
E = 320000
H = 128
G = 64
NC = 2
NS = 16
NW = NC * NS
EW = E // NW
CH = 80
NCHUNK = EW // CH
NP = 10240
STR = NP // NS

_SC_MESH = plsc.VectorSubcoreMesh(core_axis_name="c", subcore_axis_name="s")


def _zero_vmem_2d(ref, rows, cols):
    def body(r, _):
        for j in range(cols // 16):
            ref[r, pl.ds(j * 16, 16)] = jnp.zeros((16,), jnp.float32)
        return 0

    lax.fori_loop(0, rows, body, 0)


@functools.partial(
    pl.kernel,
    out_type=jax.ShapeDtypeStruct((NC, NP), jnp.float32),
    mesh=_SC_MESH,
    scratch_types=[
        pltpu.VMEM((NCHUNK, CH), jnp.int32),
        pltpu.VMEM((128,), jnp.float32),
        pltpu.VMEM((STR,), jnp.float32),
        pltpu.VMEM_SHARED((NP,), jnp.float32),
    ],
)
def _hist_sc(row_hbm, out_hbm, idx_v, ones_v, zb_v, hist_sp):
    c = lax.axis_index("c")
    s = lax.axis_index("s")
    w = c * NS + s
    for i in range(8):
        ones_v[pl.ds(i * 16, 16)] = jnp.ones((16,), jnp.float32)
    def zb(i, _):
        zb_v[pl.ds(i * 16, 16)] = jnp.zeros((16,), jnp.float32)
        return 0
    lax.fori_loop(0, STR // 16, zb, 0)
    pltpu.sync_copy(zb_v, hist_sp.at[pl.ds(s * STR, STR)])
    pltpu.sync_copy(row_hbm.at[w], idx_v)
    plsc.subcore_barrier()

    def body(ch, _):
        pltpu.sync_copy(ones_v.at[pl.ds(0, CH)],
                        hist_sp.at[idx_v.at[ch]], add=True)
        return 0

    lax.fori_loop(0, NCHUNK, body, 0)
    plsc.subcore_barrier()
    pltpu.sync_copy(hist_sp.at[pl.ds(s * STR, STR)],
                    out_hbm.at[c, pl.ds(s * STR, STR)])


@functools.partial(
    pl.kernel,
    out_type=jax.ShapeDtypeStruct((NW, NCHUNK, CH), jnp.float32),
    mesh=_SC_MESH,
    scratch_types=[
        pltpu.VMEM((NCHUNK, CH), jnp.int32),
        pltpu.VMEM((NCHUNK, CH), jnp.float32),
        pltpu.SemaphoreType.DMA,
    ],
)
def _cnorm_sc(rdeg_hbm, col_hbm, out_hbm, idx_v, o_v, sem):
    c = lax.axis_index("c")
    s = lax.axis_index("s")
    w = c * NS + s
    pltpu.sync_copy(col_hbm.at[w], idx_v)

    def body(ch, _):
        pltpu.async_copy(rdeg_hbm.at[idx_v.at[ch]], o_v.at[ch], sem).wait()
        return 0

    lax.fori_loop(0, NCHUNK, body, 0)
    pltpu.sync_copy(o_v, out_hbm.at[w])


@functools.partial(
    pl.kernel,
    out_type=jax.ShapeDtypeStruct((NC, NP, H), jnp.float32),
    mesh=_SC_MESH,
    scratch_types=[
        pltpu.VMEM((NCHUNK, CH), jnp.int32),
        pltpu.VMEM((128, H), jnp.float32),
        pltpu.VMEM_SHARED((NP, H), jnp.float32),
    ],
)
def _edge_scatter_sc(msg_hbm, row_hbm, out_hbm, idx_v, buf_v, acc_sp):
    c = lax.axis_index("c")
    s = lax.axis_index("s")
    w = c * NS + s
    _zero_vmem_2d(buf_v, 128, H)
    for k in range(STR // 128):
        pltpu.sync_copy(buf_v,
                        acc_sp.at[pl.ds(s * STR + k * 128, 128)])
    pltpu.sync_copy(row_hbm.at[w], idx_v)
    plsc.subcore_barrier()

    def body(ch, _):
        pltpu.sync_copy(msg_hbm.at[pl.ds(w * EW + ch * CH, CH)],
                        buf_v.at[pl.ds(0, CH)])
        pltpu.sync_copy(buf_v.at[pl.ds(0, CH)],
                        acc_sp.at[idx_v.at[ch]], add=True)
        return 0

    lax.fori_loop(0, NCHUNK, body, 0)
    plsc.subcore_barrier()
    pltpu.sync_copy(acc_sp.at[pl.ds(s * STR, STR)],
                    out_hbm.at[c, pl.ds(s * STR, STR)])


@functools.partial(
    pl.kernel,
    out_type=jax.ShapeDtypeStruct((NC, NP, H), jnp.float32),
    mesh=_SC_MESH,
    scratch_types=[
        pltpu.VMEM((NCHUNK, CH), jnp.int32),
        pltpu.VMEM((NCHUNK, CH), jnp.int32),
        pltpu.VMEM((128, H), jnp.float32),
        pltpu.VMEM_SHARED((NP, H), jnp.float32),
        pltpu.SemaphoreType.DMA,
    ],
)
def _spmm_sc(y_hbm, col_hbm, row_hbm, init_hbm, out_hbm,
             cidx_v, ridx_v, buf_v, acc_sp, sem):
    c = lax.axis_index("c")
    s = lax.axis_index("s")
    w = c * NS + s
    pltpu.sync_copy(init_hbm.at[c, pl.ds(s * STR, STR)],
                    acc_sp.at[pl.ds(s * STR, STR)])
    pltpu.sync_copy(col_hbm.at[w], cidx_v)
    pltpu.sync_copy(row_hbm.at[w], ridx_v)
    plsc.subcore_barrier()

    def body(ch, _):
        pltpu.async_copy(y_hbm.at[cidx_v.at[ch]],
                         buf_v.at[pl.ds(0, CH)], sem).wait()
        pltpu.sync_copy(buf_v.at[pl.ds(0, CH)],
                        acc_sp.at[ridx_v.at[ch]], add=True)
        return 0

    lax.fori_loop(0, NCHUNK, body, 0)
    plsc.subcore_barrier()
    pltpu.sync_copy(acc_sp.at[pl.ds(s * STR, STR)],
                    out_hbm.at[c, pl.ds(s * STR, STR)])


def _rdeg_body(hist_ref, out_ref):
    out_ref[...] = lax.rsqrt(1.0 + hist_ref[0, :] + hist_ref[1, :])


def _rdeg_tc(hist2):
    return pl.pallas_call(
        _rdeg_body,
        out_shape=jax.ShapeDtypeStruct((NP,), jnp.float32),
    )(hist2)


def _in_conv_body(x_ref, wn_ref, bn_ref, wc_ref, bc_ref, rd_ref, o_ref):
    h0 = jax.nn.relu(
        jnp.dot(x_ref[...], wn_ref[...], preferred_element_type=jnp.float32)
        + bn_ref[...][None, :])
    y = (jnp.dot(h0, wc_ref[...], preferred_element_type=jnp.float32)
         + bc_ref[...][None, :]) * rd_ref[...]
    o_ref[...] = y


def _in_conv_tc(x, Wn, bn, Wc0, bc0, rdeg_col):
    blk = 1024
    return pl.pallas_call(
        _in_conv_body,
        grid=(NP // blk,),
        in_specs=[
            pl.BlockSpec((blk, H), lambda i: (i, 0)),
            pl.BlockSpec((H, H), lambda i: (0, 0)),
            pl.BlockSpec((H,), lambda i: (0,)),
            pl.BlockSpec((H, H), lambda i: (0, 0)),
            pl.BlockSpec((H,), lambda i: (0,)),
            pl.BlockSpec((blk, 1), lambda i: (i, 0)),
        ],
        out_specs=pl.BlockSpec((blk, H), lambda i: (i, 0)),
        out_shape=jax.ShapeDtypeStruct((NP, H), jnp.float32),
    )(x, Wn, bn, Wc0, bc0, rdeg_col)


def _msg_body(ea_ref, we_ref, be_ref, cn_ref, o_ref):
    m = jax.nn.relu(
        jnp.dot(ea_ref[...], we_ref[...], preferred_element_type=jnp.float32)
        + be_ref[...][None, :])
    o_ref[...] = m * cn_ref[...]


def _msg_tc(edge_attr, We, be, cnorm_col):
    blk = 2000
    return pl.pallas_call(
        _msg_body,
        grid=(E // blk,),
        in_specs=[
            pl.BlockSpec((blk, 16), lambda i: (i, 0)),
            pl.BlockSpec((16, H), lambda i: (0, 0)),
            pl.BlockSpec((H,), lambda i: (0,)),
            pl.BlockSpec((blk, 1), lambda i: (i, 0)),
        ],
        out_specs=pl.BlockSpec((blk, H), lambda i: (i, 0)),
        out_shape=jax.ShapeDtypeStruct((E, H), jnp.float32),
    )(edge_attr, We, be, cnorm_col)


def _combine_body(p_ref, rd_ref, w_ref, b_ref, o_ref):
    z = (p_ref[0] + p_ref[1]) * rd_ref[...]
    u = jax.nn.relu(z) + z
    o_ref[...] = (jnp.dot(u, w_ref[...], preferred_element_type=jnp.float32)
                  + b_ref[...][None, :]) * rd_ref[...]


def _combine_matmul_tc(p, rdeg_col, W, b):
    blk = 1024
    return pl.pallas_call(
        _combine_body,
        grid=(NP // blk,),
        in_specs=[
            pl.BlockSpec((NC, blk, H), lambda i: (0, i, 0)),
            pl.BlockSpec((blk, 1), lambda i: (i, 0)),
            pl.BlockSpec((H, H), lambda i: (0, 0)),
            pl.BlockSpec((H,), lambda i: (0,)),
        ],
        out_specs=pl.BlockSpec((blk, H), lambda i: (i, 0)),
        out_shape=jax.ShapeDtypeStruct((NP, H), jnp.float32),
    )(p, rdeg_col, W, b)


def _tail_body(p_ref, rd_ref, bt_ref, w1_ref, b1_ref, wf0_ref, bf0_ref,
               wf1_ref, bf1_ref, wo_ref, bo_ref, o_ref, pool_acc, cnt_acc):
    i = pl.program_id(0)
    blk = p_ref.shape[1]

    @pl.when(i == 0)
    def _():
        pool_acc[...] = jnp.zeros_like(pool_acc)
        cnt_acc[...] = jnp.zeros_like(cnt_acc)

    z = 2.0 * (p_ref[0] + p_ref[1]) * rd_ref[...]
    gids = lax.broadcasted_iota(jnp.int32, (1, G), 1)
    oh = (bt_ref[...] == gids).astype(jnp.float32)
    pool_acc[...] += lax.dot_general(
        oh, z, (((0,), (0,)), ((), ())), preferred_element_type=jnp.float32)
    cnt_acc[...] += lax.dot_general(
        oh, jnp.ones((blk, 1), jnp.float32), (((0,), (0,)), ((), ())),
        preferred_element_type=jnp.float32)

    @pl.when(i == pl.num_programs(0) - 1)
    def _():
        pooled = pool_acc[...] / jnp.maximum(cnt_acc[...], 1.0)
        hf = jax.nn.relu(
            jnp.dot(pooled, w1_ref[...], preferred_element_type=jnp.float32)
            + b1_ref[...][None, :])
        hf = jax.nn.relu(
            jnp.dot(hf, wf0_ref[...], preferred_element_type=jnp.float32)
            + bf0_ref[...][None, :])
        hf = jax.nn.relu(
            jnp.dot(hf, wf1_ref[...], preferred_element_type=jnp.float32)
            + bf1_ref[...][None, :])
        o_ref[...] = (jnp.dot(hf, wo_ref[...],
                              preferred_element_type=jnp.float32)
                      + bo_ref[...][None, :])


def _tail_tc(p, rdeg_col, batch_col, W1, b1, Wf0, bf0, Wf1, bf1, Wo, bo):
    blk = 1024
    FH = W1.shape[1]
    return pl.pallas_call(
        _tail_body,
        grid=(NP // blk,),
        in_specs=[
            pl.BlockSpec((NC, blk, H), lambda i: (0, i, 0)),
            pl.BlockSpec((blk, 1), lambda i: (i, 0)),
            pl.BlockSpec((blk, 1), lambda i: (i, 0)),
            pl.BlockSpec((H, FH), lambda i: (0, 0)),
            pl.BlockSpec((FH,), lambda i: (0,)),
            pl.BlockSpec((FH, FH), lambda i: (0, 0)),
            pl.BlockSpec((FH,), lambda i: (0,)),
            pl.BlockSpec((FH, FH), lambda i: (0, 0)),
            pl.BlockSpec((FH,), lambda i: (0,)),
            pl.BlockSpec((FH, 1), lambda i: (0, 0)),
            pl.BlockSpec((1,), lambda i: (0,)),
        ],
        out_specs=pl.BlockSpec((G, 1), lambda i: (0, 0)),
        out_shape=jax.ShapeDtypeStruct((G, 1), jnp.float32),
        scratch_shapes=[
            pltpu.VMEM((G, H), jnp.float32),
            pltpu.VMEM((G, 1), jnp.float32),
        ],
    )(p, rdeg_col, batch_col, W1, b1, Wf0, bf0, Wf1, bf1, Wo, bo)


def kernel(x, edge_index, edge_attr, batch, parity_atoms, add_feature,
           Wn, bn, We, be, Wc0, bc0, Wc1, bc1, Wc2, bc2,
           W1, b1, Wf0, bf0, Wf1, bf1, Wo, bo):
    row = edge_index[0]
    col = edge_index[1]
    row3 = row.reshape(NW, NCHUNK, CH)
    col3 = col.reshape(NW, NCHUNK, CH)

    hist2 = _hist_sc(row3)
    rdeg = _rdeg_tc(hist2)
    cnorm = _cnorm_sc(rdeg, col3)
    rdeg_col = rdeg.reshape(NP, 1)
    cnorm_col = cnorm.reshape(E, 1)

    y = _in_conv_tc(x, Wn, bn, Wc0, bc0, rdeg_col)
    msg = _msg_tc(edge_attr, We, be, cnorm_col)
    ebar = _edge_scatter_sc(msg, row3)

    p = _spmm_sc(y, col3, row3, ebar)
    y = _combine_matmul_tc(p, rdeg_col, Wc1, bc1)
    p = _spmm_sc(y, col3, row3, ebar)
    y = _combine_matmul_tc(p, rdeg_col, Wc2, bc2)
    p = _spmm_sc(y, col3, row3, ebar)

    batch_col = jnp.concatenate(
        [batch, jnp.full((NP - N,), G, jnp.int32)]).reshape(NP, 1)
    return _tail_tc(p, rdeg_col, batch_col, W1, b1, Wf0, bf0, Wf1, bf1, Wo, bo)


# speedup vs baseline: 10.1357x; 1.5127x over previous
import functools

import jax
import jax.numpy as jnp
from jax import lax
from jax.experimental import pallas as pl
from jax.experimental.pallas import tpu as pltpu
from jax.experimental.pallas import tpu_sc as plsc

N = 10000
E = 320000
H = 128
G = 64
NC = 2
NS = 16
NW = NC * NS
EW = E // NW
CH = 40
NCHUNK = EW // CH
NP = 10240
STR = NP // NS

NBUF = 5
NRING = NCHUNK // NBUF
_DEPTH = 3
NIB = 8
_IDEPTH = 5

_SC_MESH = plsc.VectorSubcoreMesh(core_axis_name="c", subcore_axis_name="s")


def _zero_vmem_2d(ref, rows, cols):
    def body(r, _):
        for j in range(cols // 16):
            ref[r, pl.ds(j * 16, 16)] = jnp.zeros((16,), jnp.float32)
        return 0

    lax.fori_loop(0, rows, body, 0)


def _pipelined_loop(issue_idx, wait_idx, issue_gather, wait_gather,
                    issue_scatter, wait_scatter):
    for j in range(_IDEPTH):
        issue_idx(j, j)
    for j in range(_DEPTH):
        issue_gather(j, j)

    def round_body(r, _):
        for b in range(NBUF):
            ch = r * NBUF + b
            wait_gather(b)
            wait_idx()
            if b == 0:
                @pl.when(r > 0)
                def _():
                    wait_scatter()
            else:
                wait_scatter()

            @pl.when(ch + _IDEPTH < NCHUNK)
            def _():
                nxt = ch + _IDEPTH
                issue_idx(nxt, lax.rem(nxt, NIB))

            slot = (b + _DEPTH) % NBUF
            if b < NBUF - _DEPTH:
                issue_gather(ch + _DEPTH, slot)
            else:
                @pl.when(r < NRING - 1)
                def _():
                    issue_gather(ch + _DEPTH, slot)
            issue_scatter(ch, b, lax.rem(ch, NIB))
        return 0

    lax.fori_loop(0, NRING, round_body, 0)
    wait_scatter()


@functools.partial(
    pl.kernel,
    out_type=jax.ShapeDtypeStruct((NC, NP), jnp.float32),
    mesh=_SC_MESH,
    scratch_types=[
        pltpu.VMEM((NCHUNK, CH), jnp.int32),
        pltpu.VMEM((128,), jnp.float32),
        pltpu.VMEM((STR,), jnp.float32),
        pltpu.VMEM_SHARED((NP,), jnp.float32),
        pltpu.SemaphoreType.DMA,
    ],
)
def _hist_sc(row_hbm, out_hbm, idx_v, ones_v, zb_v, hist_sp, ssem):
    c = lax.axis_index("c")
    s = lax.axis_index("s")
    w = c * NS + s
    for i in range(8):
        ones_v[pl.ds(i * 16, 16)] = jnp.ones((16,), jnp.float32)

    def zb(i, _):
        zb_v[pl.ds(i * 16, 16)] = jnp.zeros((16,), jnp.float32)
        return 0

    lax.fori_loop(0, STR // 16, zb, 0)
    pltpu.sync_copy(zb_v, hist_sp.at[pl.ds(s * STR, STR)])
    pltpu.sync_copy(row_hbm.at[w], idx_v)
    plsc.subcore_barrier()

    def body(ch, _):
        @pl.when(ch >= 8)
        def _():
            pltpu.make_async_copy(out_hbm.at[c, pl.ds(0, CH)],
                                  ones_v.at[pl.ds(0, CH)], ssem).wait()
        pltpu.async_copy(ones_v.at[pl.ds(0, CH)],
                         hist_sp.at[idx_v.at[ch]], ssem, add=True)
        return 0

    lax.fori_loop(0, NCHUNK, body, 0)
    for _ in range(8):
        pltpu.make_async_copy(out_hbm.at[c, pl.ds(0, CH)],
                              ones_v.at[pl.ds(0, CH)], ssem).wait()
    plsc.subcore_barrier()
    pltpu.sync_copy(hist_sp.at[pl.ds(s * STR, STR)],
                    out_hbm.at[c, pl.ds(s * STR, STR)])


@functools.partial(
    pl.kernel,
    out_type=jax.ShapeDtypeStruct((NW, NCHUNK, CH), jnp.float32),
    mesh=_SC_MESH,
    scratch_types=[
        pltpu.VMEM((NCHUNK, CH), jnp.int32),
        pltpu.VMEM((NCHUNK, CH), jnp.float32),
        pltpu.SemaphoreType.DMA,
    ],
)
def _cnorm_sc(rdeg_hbm, col_hbm, out_hbm, idx_v, o_v, sem):
    c = lax.axis_index("c")
    s = lax.axis_index("s")
    w = c * NS + s
    pltpu.sync_copy(col_hbm.at[w], idx_v)
    for j in range(8):
        pltpu.async_copy(rdeg_hbm.at[idx_v.at[j]], o_v.at[j], sem)

    def body(ch, _):
        pltpu.make_async_copy(rdeg_hbm.at[pl.ds(0, CH)], o_v.at[ch],
                              sem).wait()

        @pl.when(ch + 8 < NCHUNK)
        def _():
            pltpu.async_copy(rdeg_hbm.at[idx_v.at[ch + 8]], o_v.at[ch + 8],
                             sem)
        return 0

    lax.fori_loop(0, NCHUNK, body, 0)
    pltpu.sync_copy(o_v, out_hbm.at[w])


@functools.partial(
    pl.kernel,
    out_type=jax.ShapeDtypeStruct((NC, NP, H), jnp.float32),
    mesh=_SC_MESH,
    scratch_types=[
        pltpu.VMEM((NIB, CH), jnp.int32),
        pltpu.VMEM((NBUF * CH, H), jnp.float32),
        pltpu.VMEM((128, H), jnp.float32),
        pltpu.VMEM_SHARED((NP, H), jnp.float32),
        pltpu.SemaphoreType.DMA,
        pltpu.SemaphoreType.DMA,
        pltpu.SemaphoreType.DMA,
    ],
)
def _edge_scatter_sc(msg_hbm, row_hbm, out_hbm, ridx_v, buf_v, zbuf_v, acc_sp,
                     isem, gsem, ssem):
    c = lax.axis_index("c")
    s = lax.axis_index("s")
    w = c * NS + s
    _zero_vmem_2d(zbuf_v, 128, H)
    for k in range(STR // 128):
        pltpu.sync_copy(zbuf_v, acc_sp.at[pl.ds(s * STR + k * 128, 128)])
    plsc.subcore_barrier()

    def ii(ch, islot):
        pltpu.async_copy(row_hbm.at[w, ch], ridx_v.at[islot], isem)

    def wi():
        pltpu.make_async_copy(row_hbm.at[0, 0], ridx_v.at[0], isem).wait()

    def ig(ch, slot):
        pltpu.async_copy(msg_hbm.at[pl.ds(w * EW + ch * CH, CH)],
                         buf_v.at[pl.ds(slot * CH, CH)], gsem)

    def wg(slot):
        pltpu.make_async_copy(msg_hbm.at[pl.ds(0, CH)],
                              buf_v.at[pl.ds(slot * CH, CH)], gsem).wait()

    def isc(ch, slot, islot):
        pltpu.async_copy(buf_v.at[pl.ds(slot * CH, CH)],
                         acc_sp.at[ridx_v.at[islot]], ssem, add=True)

    def wsc():
        pltpu.make_async_copy(msg_hbm.at[pl.ds(0, CH)],
                              buf_v.at[pl.ds(0, CH)], ssem).wait()

    _pipelined_loop(ii, wi, ig, wg, isc, wsc)
    plsc.subcore_barrier()
    pltpu.sync_copy(acc_sp.at[pl.ds(s * STR, STR)],
                    out_hbm.at[c, pl.ds(s * STR, STR)])


@functools.partial(
    pl.kernel,
    out_type=jax.ShapeDtypeStruct((NC, NP, H), jnp.float32),
    mesh=_SC_MESH,
    scratch_types=[
        pltpu.VMEM((EW,), jnp.int32),
        pltpu.VMEM((NIB, CH), jnp.int32),
        pltpu.VMEM((NBUF * CH, H), jnp.float32),
        pltpu.VMEM_SHARED((NP, H), jnp.float32),
        pltpu.SemaphoreType.DMA,
        pltpu.SemaphoreType.DMA,
        pltpu.SemaphoreType.DMA,
    ],
)
def _spmm_sc(y_hbm, col_hbm, row_hbm, init_hbm, out_hbm,
             cidx_v, ridx_v, buf_v, acc_sp, isem, gsem, ssem):
    c = lax.axis_index("c")
    s = lax.axis_index("s")
    w = c * NS + s
    pltpu.sync_copy(init_hbm.at[c, pl.ds(s * STR, STR)],
                    acc_sp.at[pl.ds(s * STR, STR)])
    pltpu.sync_copy(col_hbm.at[w], cidx_v)
    plsc.subcore_barrier()

    def ii(ch, islot):
        pltpu.async_copy(row_hbm.at[w, ch], ridx_v.at[islot], isem)

    def wi():
        pltpu.make_async_copy(row_hbm.at[0, 0], ridx_v.at[0], isem).wait()

    def ig(ch, slot):
        pltpu.async_copy(y_hbm.at[cidx_v.at[pl.ds(ch * CH, CH)]],
                         buf_v.at[pl.ds(slot * CH, CH)], gsem)

    def wg(slot):
        pltpu.make_async_copy(y_hbm.at[pl.ds(0, CH)],
                              buf_v.at[pl.ds(slot * CH, CH)], gsem).wait()

    def isc(ch, slot, islot):
        pltpu.async_copy(buf_v.at[pl.ds(slot * CH, CH)],
                         acc_sp.at[ridx_v.at[islot]], ssem, add=True)

    def wsc():
        pltpu.make_async_copy(y_hbm.at[pl.ds(0, CH)],
                              buf_v.at[pl.ds(0, CH)], ssem).wait()

    _pipelined_loop(ii, wi, ig, wg, isc, wsc)
    plsc.subcore_barrier()
    pltpu.sync_copy(acc_sp.at[pl.ds(s * STR, STR)],
                    out_hbm.at[c, pl.ds(s * STR, STR)])


def _rdeg_body(hist_ref, out_ref):
    out_ref[...] = lax.rsqrt(1.0 + hist_ref[0, :] + hist_ref[1, :])


def _rdeg_tc(hist2):
    return pl.pallas_call(
        _rdeg_body,
        out_shape=jax.ShapeDtypeStruct((NP,), jnp.float32),
    )(hist2)


def _in_conv_body(x_ref, wn_ref, bn_ref, wc_ref, bc_ref, rd_ref, o_ref):
    h0 = jax.nn.relu(
        jnp.dot(x_ref[...], wn_ref[...], preferred_element_type=jnp.float32)
        + bn_ref[...][None, :])
    y = (jnp.dot(h0, wc_ref[...], preferred_element_type=jnp.float32)
         + bc_ref[...][None, :]) * rd_ref[...]
    o_ref[...] = y


def _in_conv_tc(x, Wn, bn, Wc0, bc0, rdeg_col):
    blk = 1024
    return pl.pallas_call(
        _in_conv_body,
        grid=(NP // blk,),
        in_specs=[
            pl.BlockSpec((blk, H), lambda i: (i, 0)),
            pl.BlockSpec((H, H), lambda i: (0, 0)),
            pl.BlockSpec((H,), lambda i: (0,)),
            pl.BlockSpec((H, H), lambda i: (0, 0)),
            pl.BlockSpec((H,), lambda i: (0,)),
            pl.BlockSpec((blk, 1), lambda i: (i, 0)),
        ],
        out_specs=pl.BlockSpec((blk, H), lambda i: (i, 0)),
        out_shape=jax.ShapeDtypeStruct((NP, H), jnp.float32),
    )(x, Wn, bn, Wc0, bc0, rdeg_col)


def _msg_body(ea_ref, we_ref, be_ref, cn_ref, o_ref):
    m = jax.nn.relu(
        jnp.dot(ea_ref[...], we_ref[...], preferred_element_type=jnp.float32)
        + be_ref[...][None, :])
    o_ref[...] = m * cn_ref[...]


def _msg_tc(edge_attr, We, be, cnorm_col):
    blk = 2000
    return pl.pallas_call(
        _msg_body,
        grid=(E // blk,),
        in_specs=[
            pl.BlockSpec((blk, 16), lambda i: (i, 0)),
            pl.BlockSpec((16, H), lambda i: (0, 0)),
            pl.BlockSpec((H,), lambda i: (0,)),
            pl.BlockSpec((blk, 1), lambda i: (i, 0)),
        ],
        out_specs=pl.BlockSpec((blk, H), lambda i: (i, 0)),
        out_shape=jax.ShapeDtypeStruct((E, H), jnp.float32),
    )(edge_attr, We, be, cnorm_col)


def _combine_body(p_ref, rd_ref, w_ref, b_ref, o_ref):
    z = (p_ref[0] + p_ref[1]) * rd_ref[...]
    u = jax.nn.relu(z) + z
    o_ref[...] = (jnp.dot(u, w_ref[...], preferred_element_type=jnp.float32)
                  + b_ref[...][None, :]) * rd_ref[...]


def _combine_matmul_tc(p, rdeg_col, W, b):
    blk = 1024
    return pl.pallas_call(
        _combine_body,
        grid=(NP // blk,),
        in_specs=[
            pl.BlockSpec((NC, blk, H), lambda i: (0, i, 0)),
            pl.BlockSpec((blk, 1), lambda i: (i, 0)),
            pl.BlockSpec((H, H), lambda i: (0, 0)),
            pl.BlockSpec((H,), lambda i: (0,)),
        ],
        out_specs=pl.BlockSpec((blk, H), lambda i: (i, 0)),
        out_shape=jax.ShapeDtypeStruct((NP, H), jnp.float32),
    )(p, rdeg_col, W, b)


def _tail_body(p_ref, rd_ref, bt_ref, w1_ref, b1_ref, wf0_ref, bf0_ref,
               wf1_ref, bf1_ref, wo_ref, bo_ref, o_ref, pool_acc, cnt_acc):
    i = pl.program_id(0)
    blk = p_ref.shape[1]

    @pl.when(i == 0)
    def _():
        pool_acc[...] = jnp.zeros_like(pool_acc)
        cnt_acc[...] = jnp.zeros_like(cnt_acc)

    z = 2.0 * (p_ref[0] + p_ref[1]) * rd_ref[...]
    gids = lax.broadcasted_iota(jnp.int32, (1, G), 1)
    oh = (bt_ref[...] == gids).astype(jnp.float32)
    pool_acc[...] += lax.dot_general(
        oh, z, (((0,), (0,)), ((), ())), preferred_element_type=jnp.float32)
    cnt_acc[...] += lax.dot_general(
        oh, jnp.ones((blk, 1), jnp.float32), (((0,), (0,)), ((), ())),
        preferred_element_type=jnp.float32)

    @pl.when(i == pl.num_programs(0) - 1)
    def _():
        pooled = pool_acc[...] / jnp.maximum(cnt_acc[...], 1.0)
        hf = jax.nn.relu(
            jnp.dot(pooled, w1_ref[...], preferred_element_type=jnp.float32)
            + b1_ref[...][None, :])
        hf = jax.nn.relu(
            jnp.dot(hf, wf0_ref[...], preferred_element_type=jnp.float32)
            + bf0_ref[...][None, :])
        hf = jax.nn.relu(
            jnp.dot(hf, wf1_ref[...], preferred_element_type=jnp.float32)
            + bf1_ref[...][None, :])
        o_ref[...] = (jnp.dot(hf, wo_ref[...],
                              preferred_element_type=jnp.float32)
                      + bo_ref[...][None, :])


def _tail_tc(p, rdeg_col, batch_col, W1, b1, Wf0, bf0, Wf1, bf1, Wo, bo):
    blk = 1024
    FH = W1.shape[1]
    return pl.pallas_call(
        _tail_body,
        grid=(NP // blk,),
        in_specs=[
            pl.BlockSpec((NC, blk, H), lambda i: (0, i, 0)),
            pl.BlockSpec((blk, 1), lambda i: (i, 0)),
            pl.BlockSpec((blk, 1), lambda i: (i, 0)),
            pl.BlockSpec((H, FH), lambda i: (0, 0)),
            pl.BlockSpec((FH,), lambda i: (0,)),
            pl.BlockSpec((FH, FH), lambda i: (0, 0)),
            pl.BlockSpec((FH,), lambda i: (0,)),
            pl.BlockSpec((FH, FH), lambda i: (0, 0)),
            pl.BlockSpec((FH,), lambda i: (0,)),
            pl.BlockSpec((FH, 1), lambda i: (0, 0)),
            pl.BlockSpec((1,), lambda i: (0,)),
        ],
        out_specs=pl.BlockSpec((G, 1), lambda i: (0, 0)),
        out_shape=jax.ShapeDtypeStruct((G, 1), jnp.float32),
        scratch_shapes=[
            pltpu.VMEM((G, H), jnp.float32),
            pltpu.VMEM((G, 1), jnp.float32),
        ],
    )(p, rdeg_col, batch_col, W1, b1, Wf0, bf0, Wf1, bf1, Wo, bo)


def kernel(x, edge_index, edge_attr, batch, parity_atoms, add_feature,
           Wn, bn, We, be, Wc0, bc0, Wc1, bc1, Wc2, bc2,
           W1, b1, Wf0, bf0, Wf1, bf1, Wo, bo):
    row = edge_index[0]
    col = edge_index[1]
    row3 = row.reshape(NW, NCHUNK, CH)
    col3 = col.reshape(NW, NCHUNK, CH)
    col2 = col.reshape(NW, EW)

    hist2 = _hist_sc(row3)
    rdeg = _rdeg_tc(hist2)
    cnorm = _cnorm_sc(rdeg, col3)
    rdeg_col = rdeg.reshape(NP, 1)
    cnorm_col = cnorm.reshape(E, 1)

    y = _in_conv_tc(x, Wn, bn, Wc0, bc0, rdeg_col)
    msg = _msg_tc(edge_attr, We, be, cnorm_col)
    ebar = _edge_scatter_sc(msg, row3)

    p = _spmm_sc(y, col2, row3, ebar)
    y = _combine_matmul_tc(p, rdeg_col, Wc1, bc1)
    p = _spmm_sc(y, col2, row3, ebar)
    y = _combine_matmul_tc(p, rdeg_col, Wc2, bc2)
    p = _spmm_sc(y, col2, row3, ebar)

    batch_col = jnp.concatenate(
        [batch, jnp.full((NP - N,), G, jnp.int32)]).reshape(NP, 1)
    return _tail_tc(p, rdeg_col, batch_col, W1, b1, Wf0, bf0, Wf1, bf1, Wo, bo)


# speedup vs baseline: 10.2828x; 1.0145x over previous
import functools

import jax
import jax.numpy as jnp
from jax import lax
from jax.experimental import pallas as pl
from jax.experimental.pallas import tpu as pltpu
from jax.experimental.pallas import tpu_sc as plsc

N = 10000
E = 320000
H = 128
G = 64
NC = 2
NS = 16
NW = NC * NS
EW = E // NW
CH = 40
NCHUNK = EW // CH
NP = 10240
STR = NP // NS

NBUF = 5
NRING = NCHUNK // NBUF
_DEPTH = 3
NIB = 8
_IDEPTH = 5

_SC_MESH = plsc.VectorSubcoreMesh(core_axis_name="c", subcore_axis_name="s")


def _zero_vmem_2d(ref, rows, cols):
    def body(r, _):
        for j in range(cols // 16):
            ref[r, pl.ds(j * 16, 16)] = jnp.zeros((16,), jnp.float32)
        return 0

    lax.fori_loop(0, rows, body, 0)


def _pipelined_loop(issue_idx, wait_idx, issue_gather, wait_gather,
                    issue_scatter, wait_scatter):
    for j in range(_IDEPTH):
        issue_idx(j, j)
    for j in range(_DEPTH):
        issue_gather(j, j)

    def round_body(r, _):
        for b in range(NBUF):
            ch = r * NBUF + b
            wait_gather(b)
            wait_idx()
            if b == 0:
                @pl.when(r > 0)
                def _():
                    wait_scatter()
            else:
                wait_scatter()

            @pl.when(ch + _IDEPTH < NCHUNK)
            def _():
                nxt = ch + _IDEPTH
                issue_idx(nxt, lax.rem(nxt, NIB))

            slot = (b + _DEPTH) % NBUF
            if b < NBUF - _DEPTH:
                issue_gather(ch + _DEPTH, slot)
            else:
                @pl.when(r < NRING - 1)
                def _():
                    issue_gather(ch + _DEPTH, slot)
            issue_scatter(ch, b, lax.rem(ch, NIB))
        return 0

    lax.fori_loop(0, NRING, round_body, 0)
    wait_scatter()


@functools.partial(
    pl.kernel,
    out_type=jax.ShapeDtypeStruct((NC, NP), jnp.float32),
    mesh=_SC_MESH,
    scratch_types=[
        pltpu.VMEM((NCHUNK, CH), jnp.int32),
        pltpu.VMEM((128,), jnp.float32),
        pltpu.VMEM((STR,), jnp.float32),
        pltpu.VMEM_SHARED((NP,), jnp.float32),
        pltpu.SemaphoreType.DMA,
    ],
)
def _hist_sc(row_hbm, out_hbm, idx_v, ones_v, zb_v, hist_sp, ssem):
    c = lax.axis_index("c")
    s = lax.axis_index("s")
    w = c * NS + s
    for i in range(8):
        ones_v[pl.ds(i * 16, 16)] = jnp.ones((16,), jnp.float32)

    def zb(i, _):
        zb_v[pl.ds(i * 16, 16)] = jnp.zeros((16,), jnp.float32)
        return 0

    lax.fori_loop(0, STR // 16, zb, 0)
    pltpu.sync_copy(zb_v, hist_sp.at[pl.ds(s * STR, STR)])
    pltpu.sync_copy(row_hbm.at[w], idx_v)
    plsc.subcore_barrier()

    def body(ch, _):
        @pl.when(ch >= 8)
        def _():
            pltpu.make_async_copy(out_hbm.at[c, pl.ds(0, CH)],
                                  ones_v.at[pl.ds(0, CH)], ssem).wait()
        pltpu.async_copy(ones_v.at[pl.ds(0, CH)],
                         hist_sp.at[idx_v.at[ch]], ssem, add=True)
        return 0

    lax.fori_loop(0, NCHUNK, body, 0)
    for _ in range(8):
        pltpu.make_async_copy(out_hbm.at[c, pl.ds(0, CH)],
                              ones_v.at[pl.ds(0, CH)], ssem).wait()
    plsc.subcore_barrier()
    pltpu.sync_copy(hist_sp.at[pl.ds(s * STR, STR)],
                    out_hbm.at[c, pl.ds(s * STR, STR)])


@functools.partial(
    pl.kernel,
    out_type=jax.ShapeDtypeStruct((NW, NCHUNK, CH), jnp.float32),
    mesh=_SC_MESH,
    scratch_types=[
        pltpu.VMEM((NCHUNK, CH), jnp.int32),
        pltpu.VMEM((NCHUNK, CH), jnp.float32),
        pltpu.VMEM_SHARED((NP,), jnp.float32),
        pltpu.SemaphoreType.DMA,
    ],
)
def _cnorm_sc(rdeg_hbm, col_hbm, out_hbm, idx_v, o_v, rdeg_sp, sem):
    c = lax.axis_index("c")
    s = lax.axis_index("s")
    w = c * NS + s

    @pl.when(s == 0)
    def _():
        pltpu.sync_copy(rdeg_hbm, rdeg_sp)
    pltpu.sync_copy(col_hbm.at[w], idx_v)
    plsc.subcore_barrier()
    for j in range(8):
        pltpu.async_copy(rdeg_sp.at[idx_v.at[j]], o_v.at[j], sem)

    def body(ch, _):
        pltpu.make_async_copy(rdeg_hbm.at[pl.ds(0, CH)], o_v.at[ch],
                              sem).wait()

        @pl.when(ch + 8 < NCHUNK)
        def _():
            pltpu.async_copy(rdeg_sp.at[idx_v.at[ch + 8]], o_v.at[ch + 8],
                             sem)
        return 0

    lax.fori_loop(0, NCHUNK, body, 0)
    pltpu.sync_copy(o_v, out_hbm.at[w])


@functools.partial(
    pl.kernel,
    out_type=[jax.ShapeDtypeStruct((NC, NP, H), jnp.float32),
              jax.ShapeDtypeStruct((NC, NP, H), jnp.float32)],
    mesh=_SC_MESH,
    scratch_types=[
        pltpu.VMEM((EW,), jnp.int32),
        pltpu.VMEM((NIB, CH), jnp.int32),
        pltpu.VMEM((NBUF * CH, H), jnp.float32),
        pltpu.VMEM((64, H), jnp.float32),
        pltpu.VMEM_SHARED((NP, H), jnp.float32),
        pltpu.SemaphoreType.DMA,
        pltpu.SemaphoreType.DMA,
        pltpu.SemaphoreType.DMA,
    ],
)
def _edge_spmm_sc(msg_hbm, y_hbm, col_hbm, row_hbm, ebar_hbm, p_hbm,
                  cidx_v, ridx_v, buf_v, zbuf_v, acc_sp, isem, gsem, ssem):
    c = lax.axis_index("c")
    s = lax.axis_index("s")
    w = c * NS + s
    _zero_vmem_2d(zbuf_v, 64, H)
    for k in range(STR // 64):
        pltpu.sync_copy(zbuf_v, acc_sp.at[pl.ds(s * STR + k * 64, 64)])
    pltpu.sync_copy(col_hbm.at[w], cidx_v)
    plsc.subcore_barrier()

    def ii(ch, islot):
        pltpu.async_copy(row_hbm.at[w, ch], ridx_v.at[islot], isem)

    def wi():
        pltpu.make_async_copy(row_hbm.at[0, 0], ridx_v.at[0], isem).wait()

    def ig_msg(ch, slot):
        pltpu.async_copy(msg_hbm.at[pl.ds(w * EW + ch * CH, CH)],
                         buf_v.at[pl.ds(slot * CH, CH)], gsem)

    def ig_y(ch, slot):
        pltpu.async_copy(y_hbm.at[cidx_v.at[pl.ds(ch * CH, CH)]],
                         buf_v.at[pl.ds(slot * CH, CH)], gsem)

    def wg(slot):
        pltpu.make_async_copy(msg_hbm.at[pl.ds(0, CH)],
                              buf_v.at[pl.ds(slot * CH, CH)], gsem).wait()

    def isc(ch, slot, islot):
        pltpu.async_copy(buf_v.at[pl.ds(slot * CH, CH)],
                         acc_sp.at[ridx_v.at[islot]], ssem, add=True)

    def wsc():
        pltpu.make_async_copy(msg_hbm.at[pl.ds(0, CH)],
                              buf_v.at[pl.ds(0, CH)], ssem).wait()

    _pipelined_loop(ii, wi, ig_msg, wg, isc, wsc)
    plsc.subcore_barrier()
    pltpu.sync_copy(acc_sp.at[pl.ds(s * STR, STR)],
                    ebar_hbm.at[c, pl.ds(s * STR, STR)])
    plsc.subcore_barrier()
    _pipelined_loop(ii, wi, ig_y, wg, isc, wsc)
    plsc.subcore_barrier()
    pltpu.sync_copy(acc_sp.at[pl.ds(s * STR, STR)],
                    p_hbm.at[c, pl.ds(s * STR, STR)])


@functools.partial(
    pl.kernel,
    out_type=jax.ShapeDtypeStruct((NC, NP, H), jnp.float32),
    mesh=_SC_MESH,
    scratch_types=[
        pltpu.VMEM((EW,), jnp.int32),
        pltpu.VMEM((NIB, CH), jnp.int32),
        pltpu.VMEM((NBUF * CH, H), jnp.float32),
        pltpu.VMEM_SHARED((NP, H), jnp.float32),
        pltpu.SemaphoreType.DMA,
        pltpu.SemaphoreType.DMA,
        pltpu.SemaphoreType.DMA,
    ],
)
def _spmm_sc(y_hbm, col_hbm, row_hbm, init_hbm, out_hbm,
             cidx_v, ridx_v, buf_v, acc_sp, isem, gsem, ssem):
    c = lax.axis_index("c")
    s = lax.axis_index("s")
    w = c * NS + s
    pltpu.sync_copy(init_hbm.at[c, pl.ds(s * STR, STR)],
                    acc_sp.at[pl.ds(s * STR, STR)])
    pltpu.sync_copy(col_hbm.at[w], cidx_v)
    plsc.subcore_barrier()

    def ii(ch, islot):
        pltpu.async_copy(row_hbm.at[w, ch], ridx_v.at[islot], isem)

    def wi():
        pltpu.make_async_copy(row_hbm.at[0, 0], ridx_v.at[0], isem).wait()

    def ig(ch, slot):
        pltpu.async_copy(y_hbm.at[cidx_v.at[pl.ds(ch * CH, CH)]],
                         buf_v.at[pl.ds(slot * CH, CH)], gsem)

    def wg(slot):
        pltpu.make_async_copy(y_hbm.at[pl.ds(0, CH)],
                              buf_v.at[pl.ds(slot * CH, CH)], gsem).wait()

    def isc(ch, slot, islot):
        pltpu.async_copy(buf_v.at[pl.ds(slot * CH, CH)],
                         acc_sp.at[ridx_v.at[islot]], ssem, add=True)

    def wsc():
        pltpu.make_async_copy(y_hbm.at[pl.ds(0, CH)],
                              buf_v.at[pl.ds(0, CH)], ssem).wait()

    _pipelined_loop(ii, wi, ig, wg, isc, wsc)
    plsc.subcore_barrier()
    pltpu.sync_copy(acc_sp.at[pl.ds(s * STR, STR)],
                    out_hbm.at[c, pl.ds(s * STR, STR)])


def _rdeg_body(hist_ref, out_ref):
    out_ref[...] = lax.rsqrt(1.0 + hist_ref[0, :] + hist_ref[1, :])


def _rdeg_tc(hist2):
    return pl.pallas_call(
        _rdeg_body,
        out_shape=jax.ShapeDtypeStruct((NP,), jnp.float32),
    )(hist2)


def _in_conv_body(x_ref, wn_ref, bn_ref, wc_ref, bc_ref, rd_ref, o_ref):
    h0 = jax.nn.relu(
        jnp.dot(x_ref[...], wn_ref[...], preferred_element_type=jnp.float32)
        + bn_ref[...][None, :])
    y = (jnp.dot(h0, wc_ref[...], preferred_element_type=jnp.float32)
         + bc_ref[...][None, :]) * rd_ref[...]
    o_ref[...] = y


def _in_conv_tc(x, Wn, bn, Wc0, bc0, rdeg_col):
    blk = 1024
    return pl.pallas_call(
        _in_conv_body,
        grid=(NP // blk,),
        in_specs=[
            pl.BlockSpec((blk, H), lambda i: (i, 0)),
            pl.BlockSpec((H, H), lambda i: (0, 0)),
            pl.BlockSpec((H,), lambda i: (0,)),
            pl.BlockSpec((H, H), lambda i: (0, 0)),
            pl.BlockSpec((H,), lambda i: (0,)),
            pl.BlockSpec((blk, 1), lambda i: (i, 0)),
        ],
        out_specs=pl.BlockSpec((blk, H), lambda i: (i, 0)),
        out_shape=jax.ShapeDtypeStruct((NP, H), jnp.float32),
    )(x, Wn, bn, Wc0, bc0, rdeg_col)


def _msg_body(ea_ref, we_ref, be_ref, cn_ref, o_ref):
    m = jax.nn.relu(
        jnp.dot(ea_ref[...], we_ref[...], preferred_element_type=jnp.float32)
        + be_ref[...][None, :])
    o_ref[...] = m * cn_ref[...]


def _msg_tc(edge_attr, We, be, cnorm_col):
    blk = 2000
    return pl.pallas_call(
        _msg_body,
        grid=(E // blk,),
        in_specs=[
            pl.BlockSpec((blk, 16), lambda i: (i, 0)),
            pl.BlockSpec((16, H), lambda i: (0, 0)),
            pl.BlockSpec((H,), lambda i: (0,)),
            pl.BlockSpec((blk, 1), lambda i: (i, 0)),
        ],
        out_specs=pl.BlockSpec((blk, H), lambda i: (i, 0)),
        out_shape=jax.ShapeDtypeStruct((E, H), jnp.float32),
    )(edge_attr, We, be, cnorm_col)


def _combine_body(p_ref, rd_ref, w_ref, b_ref, o_ref):
    z = (p_ref[0] + p_ref[1]) * rd_ref[...]
    u = jax.nn.relu(z) + z
    o_ref[...] = (jnp.dot(u, w_ref[...], preferred_element_type=jnp.float32)
                  + b_ref[...][None, :]) * rd_ref[...]


def _combine_matmul_tc(p, rdeg_col, W, b):
    blk = 1024
    return pl.pallas_call(
        _combine_body,
        grid=(NP // blk,),
        in_specs=[
            pl.BlockSpec((NC, blk, H), lambda i: (0, i, 0)),
            pl.BlockSpec((blk, 1), lambda i: (i, 0)),
            pl.BlockSpec((H, H), lambda i: (0, 0)),
            pl.BlockSpec((H,), lambda i: (0,)),
        ],
        out_specs=pl.BlockSpec((blk, H), lambda i: (i, 0)),
        out_shape=jax.ShapeDtypeStruct((NP, H), jnp.float32),
    )(p, rdeg_col, W, b)


def _tail_body(p_ref, rd_ref, bt_ref, w1_ref, b1_ref, wf0_ref, bf0_ref,
               wf1_ref, bf1_ref, wo_ref, bo_ref, o_ref, pool_acc, cnt_acc):
    i = pl.program_id(0)
    blk = p_ref.shape[1]

    @pl.when(i == 0)
    def _():
        pool_acc[...] = jnp.zeros_like(pool_acc)
        cnt_acc[...] = jnp.zeros_like(cnt_acc)

    z = 2.0 * (p_ref[0] + p_ref[1]) * rd_ref[...]
    gids = lax.broadcasted_iota(jnp.int32, (1, G), 1)
    oh = (bt_ref[...] == gids).astype(jnp.float32)
    pool_acc[...] += lax.dot_general(
        oh, z, (((0,), (0,)), ((), ())), preferred_element_type=jnp.float32)
    cnt_acc[...] += lax.dot_general(
        oh, jnp.ones((blk, 1), jnp.float32), (((0,), (0,)), ((), ())),
        preferred_element_type=jnp.float32)

    @pl.when(i == pl.num_programs(0) - 1)
    def _():
        pooled = pool_acc[...] / jnp.maximum(cnt_acc[...], 1.0)
        hf = jax.nn.relu(
            jnp.dot(pooled, w1_ref[...], preferred_element_type=jnp.float32)
            + b1_ref[...][None, :])
        hf = jax.nn.relu(
            jnp.dot(hf, wf0_ref[...], preferred_element_type=jnp.float32)
            + bf0_ref[...][None, :])
        hf = jax.nn.relu(
            jnp.dot(hf, wf1_ref[...], preferred_element_type=jnp.float32)
            + bf1_ref[...][None, :])
        o_ref[...] = (jnp.dot(hf, wo_ref[...],
                              preferred_element_type=jnp.float32)
                      + bo_ref[...][None, :])


def _tail_tc(p, rdeg_col, batch_col, W1, b1, Wf0, bf0, Wf1, bf1, Wo, bo):
    blk = 1024
    FH = W1.shape[1]
    return pl.pallas_call(
        _tail_body,
        grid=(NP // blk,),
        in_specs=[
            pl.BlockSpec((NC, blk, H), lambda i: (0, i, 0)),
            pl.BlockSpec((blk, 1), lambda i: (i, 0)),
            pl.BlockSpec((blk, 1), lambda i: (i, 0)),
            pl.BlockSpec((H, FH), lambda i: (0, 0)),
            pl.BlockSpec((FH,), lambda i: (0,)),
            pl.BlockSpec((FH, FH), lambda i: (0, 0)),
            pl.BlockSpec((FH,), lambda i: (0,)),
            pl.BlockSpec((FH, FH), lambda i: (0, 0)),
            pl.BlockSpec((FH,), lambda i: (0,)),
            pl.BlockSpec((FH, 1), lambda i: (0, 0)),
            pl.BlockSpec((1,), lambda i: (0,)),
        ],
        out_specs=pl.BlockSpec((G, 1), lambda i: (0, 0)),
        out_shape=jax.ShapeDtypeStruct((G, 1), jnp.float32),
        scratch_shapes=[
            pltpu.VMEM((G, H), jnp.float32),
            pltpu.VMEM((G, 1), jnp.float32),
        ],
    )(p, rdeg_col, batch_col, W1, b1, Wf0, bf0, Wf1, bf1, Wo, bo)


def kernel(x, edge_index, edge_attr, batch, parity_atoms, add_feature,
           Wn, bn, We, be, Wc0, bc0, Wc1, bc1, Wc2, bc2,
           W1, b1, Wf0, bf0, Wf1, bf1, Wo, bo):
    row = edge_index[0]
    col = edge_index[1]
    row3 = row.reshape(NW, NCHUNK, CH)
    col3 = col.reshape(NW, NCHUNK, CH)
    col2 = col.reshape(NW, EW)

    hist2 = _hist_sc(row3)
    rdeg = _rdeg_tc(hist2)
    cnorm = _cnorm_sc(rdeg, col3)
    rdeg_col = rdeg.reshape(NP, 1)
    cnorm_col = cnorm.reshape(E, 1)

    y = _in_conv_tc(x, Wn, bn, Wc0, bc0, rdeg_col)
    msg = _msg_tc(edge_attr, We, be, cnorm_col)
    ebar, p = _edge_spmm_sc(msg, y, col2, row3)

    y = _combine_matmul_tc(p, rdeg_col, Wc1, bc1)
    p = _spmm_sc(y, col2, row3, ebar)
    y = _combine_matmul_tc(p, rdeg_col, Wc2, bc2)
    p = _spmm_sc(y, col2, row3, ebar)

    batch_col = jnp.concatenate(
        [batch, jnp.full((NP - N,), G, jnp.int32)]).reshape(NP, 1)
    return _tail_tc(p, rdeg_col, batch_col, W1, b1, Wf0, bf0, Wf1, bf1, Wo, bo)


# speedup vs baseline: 10.6768x; 1.0383x over previous
import functools

import jax
import jax.numpy as jnp
from jax import lax
from jax.experimental import pallas as pl
from jax.experimental.pallas import tpu as pltpu
from jax.experimental.pallas import tpu_sc as plsc

N = 10000
E = 320000
H = 128
G = 64
NC = 2
NS = 16
NW = NC * NS
EW = E // NW
CH = 40
NCHUNK = EW // CH
NP = 10240
STR = NP // NS

NBUF = 7
_DEPTH = 5
NIB = 8
_IDEPTH = 5

_SC_MESH = plsc.VectorSubcoreMesh(core_axis_name="c", subcore_axis_name="s")


def _zero_vmem_2d(ref, rows, cols):
    def body(r, _):
        for j in range(cols // 16):
            ref[r, pl.ds(j * 16, 16)] = jnp.zeros((16,), jnp.float32)
        return 0

    lax.fori_loop(0, rows, body, 0)


def _pipelined_loop(issue_idx, wait_idx, issue_gather, wait_gather,
                    issue_scatter, wait_scatter):
    for j in range(_IDEPTH):
        issue_idx(j, j)
    for j in range(_DEPTH):
        issue_gather(j, j)

    def step(ch, _):
        wait_gather(0)
        wait_idx()

        @pl.when(ch > 0)
        def _():
            wait_scatter()

        @pl.when(ch + _IDEPTH < NCHUNK)
        def _():
            nxt = ch + _IDEPTH
            issue_idx(nxt, lax.rem(nxt, NIB))

        @pl.when(ch + _DEPTH < NCHUNK)
        def _():
            nxt = ch + _DEPTH
            issue_gather(nxt, lax.rem(nxt, NBUF))
        issue_scatter(ch, lax.rem(ch, NBUF), lax.rem(ch, NIB))
        return 0

    lax.fori_loop(0, NCHUNK, step, 0)
    wait_scatter()


@functools.partial(
    pl.kernel,
    out_type=jax.ShapeDtypeStruct((NC, NP), jnp.float32),
    mesh=_SC_MESH,
    scratch_types=[
        pltpu.VMEM((NCHUNK, CH), jnp.int32),
        pltpu.VMEM((128,), jnp.float32),
        pltpu.VMEM((STR,), jnp.float32),
        pltpu.VMEM_SHARED((NP,), jnp.float32),
        pltpu.SemaphoreType.DMA,
    ],
)
def _hist_sc(row_hbm, out_hbm, idx_v, ones_v, zb_v, hist_sp, ssem):
    c = lax.axis_index("c")
    s = lax.axis_index("s")
    w = c * NS + s
    for i in range(8):
        ones_v[pl.ds(i * 16, 16)] = jnp.ones((16,), jnp.float32)

    def zb(i, _):
        zb_v[pl.ds(i * 16, 16)] = jnp.zeros((16,), jnp.float32)
        return 0

    lax.fori_loop(0, STR // 16, zb, 0)
    pltpu.sync_copy(zb_v, hist_sp.at[pl.ds(s * STR, STR)])
    pltpu.sync_copy(row_hbm.at[w], idx_v)
    plsc.subcore_barrier()

    def body(ch, _):
        @pl.when(ch >= 8)
        def _():
            pltpu.make_async_copy(out_hbm.at[c, pl.ds(0, CH)],
                                  ones_v.at[pl.ds(0, CH)], ssem).wait()
        pltpu.async_copy(ones_v.at[pl.ds(0, CH)],
                         hist_sp.at[idx_v.at[ch]], ssem, add=True)
        return 0

    lax.fori_loop(0, NCHUNK, body, 0)
    for _ in range(8):
        pltpu.make_async_copy(out_hbm.at[c, pl.ds(0, CH)],
                              ones_v.at[pl.ds(0, CH)], ssem).wait()
    plsc.subcore_barrier()
    pltpu.sync_copy(hist_sp.at[pl.ds(s * STR, STR)],
                    out_hbm.at[c, pl.ds(s * STR, STR)])


@functools.partial(
    pl.kernel,
    out_type=jax.ShapeDtypeStruct((NW, NCHUNK, CH), jnp.float32),
    mesh=_SC_MESH,
    scratch_types=[
        pltpu.VMEM((NCHUNK, CH), jnp.int32),
        pltpu.VMEM((NCHUNK, CH), jnp.float32),
        pltpu.VMEM_SHARED((NP,), jnp.float32),
        pltpu.SemaphoreType.DMA,
    ],
)
def _cnorm_sc(rdeg_hbm, col_hbm, out_hbm, idx_v, o_v, rdeg_sp, sem):
    c = lax.axis_index("c")
    s = lax.axis_index("s")
    w = c * NS + s

    @pl.when(s == 0)
    def _():
        pltpu.sync_copy(rdeg_hbm, rdeg_sp)
    pltpu.sync_copy(col_hbm.at[w], idx_v)
    plsc.subcore_barrier()
    for j in range(8):
        pltpu.async_copy(rdeg_sp.at[idx_v.at[j]], o_v.at[j], sem)

    def body(ch, _):
        pltpu.make_async_copy(rdeg_hbm.at[pl.ds(0, CH)], o_v.at[ch],
                              sem).wait()

        @pl.when(ch + 8 < NCHUNK)
        def _():
            pltpu.async_copy(rdeg_sp.at[idx_v.at[ch + 8]], o_v.at[ch + 8],
                             sem)
        return 0

    lax.fori_loop(0, NCHUNK, body, 0)
    pltpu.sync_copy(o_v, out_hbm.at[w])


@functools.partial(
    pl.kernel,
    out_type=[jax.ShapeDtypeStruct((NC, NP, H), jnp.float32),
              jax.ShapeDtypeStruct((NC, NP, H), jnp.float32)],
    mesh=_SC_MESH,
    scratch_types=[
        pltpu.VMEM((EW,), jnp.int32),
        pltpu.VMEM((NIB, CH), jnp.int32),
        pltpu.VMEM((NBUF * CH, H), jnp.float32),
        pltpu.VMEM((16, H), jnp.float32),
        pltpu.VMEM_SHARED((NP, H), jnp.float32),
        pltpu.SemaphoreType.DMA,
        pltpu.SemaphoreType.DMA,
        pltpu.SemaphoreType.DMA,
    ],
)
def _edge_spmm_sc(msg_hbm, y_hbm, col_hbm, row_hbm, ebar_hbm, p_hbm,
                  cidx_v, ridx_v, buf_v, zbuf_v, acc_sp, isem, gsem, ssem):
    c = lax.axis_index("c")
    s = lax.axis_index("s")
    w = c * NS + s
    _zero_vmem_2d(zbuf_v, 16, H)

    def zcp(k, _):
        pltpu.sync_copy(zbuf_v, acc_sp.at[pl.ds(s * STR + k * 16, 16)])
        return 0

    lax.fori_loop(0, STR // 16, zcp, 0)
    pltpu.sync_copy(col_hbm.at[w], cidx_v)
    plsc.subcore_barrier()

    def ii(ch, islot):
        pltpu.async_copy(row_hbm.at[w, ch], ridx_v.at[islot], isem)

    def wi():
        pltpu.make_async_copy(row_hbm.at[0, 0], ridx_v.at[0], isem).wait()

    def ig_msg(ch, slot):
        pltpu.async_copy(msg_hbm.at[pl.ds(w * EW + ch * CH, CH)],
                         buf_v.at[pl.ds(slot * CH, CH)], gsem)

    def ig_y(ch, slot):
        pltpu.async_copy(y_hbm.at[cidx_v.at[pl.ds(ch * CH, CH)]],
                         buf_v.at[pl.ds(slot * CH, CH)], gsem)

    def wg(slot):
        pltpu.make_async_copy(msg_hbm.at[pl.ds(0, CH)],
                              buf_v.at[pl.ds(slot * CH, CH)], gsem).wait()

    def isc(ch, slot, islot):
        pltpu.async_copy(buf_v.at[pl.ds(slot * CH, CH)],
                         acc_sp.at[ridx_v.at[islot]], ssem, add=True)

    def wsc():
        pltpu.make_async_copy(msg_hbm.at[pl.ds(0, CH)],
                              buf_v.at[pl.ds(0, CH)], ssem).wait()

    _pipelined_loop(ii, wi, ig_msg, wg, isc, wsc)
    plsc.subcore_barrier()
    pltpu.sync_copy(acc_sp.at[pl.ds(s * STR, STR)],
                    ebar_hbm.at[c, pl.ds(s * STR, STR)])
    plsc.subcore_barrier()
    _pipelined_loop(ii, wi, ig_y, wg, isc, wsc)
    plsc.subcore_barrier()
    pltpu.sync_copy(acc_sp.at[pl.ds(s * STR, STR)],
                    p_hbm.at[c, pl.ds(s * STR, STR)])


@functools.partial(
    pl.kernel,
    out_type=jax.ShapeDtypeStruct((NC, NP, H), jnp.float32),
    mesh=_SC_MESH,
    scratch_types=[
        pltpu.VMEM((EW,), jnp.int32),
        pltpu.VMEM((NIB, CH), jnp.int32),
        pltpu.VMEM((NBUF * CH, H), jnp.float32),
        pltpu.VMEM_SHARED((NP, H), jnp.float32),
        pltpu.SemaphoreType.DMA,
        pltpu.SemaphoreType.DMA,
        pltpu.SemaphoreType.DMA,
    ],
)
def _spmm_sc(y_hbm, col_hbm, row_hbm, init_hbm, out_hbm,
             cidx_v, ridx_v, buf_v, acc_sp, isem, gsem, ssem):
    c = lax.axis_index("c")
    s = lax.axis_index("s")
    w = c * NS + s
    pltpu.sync_copy(init_hbm.at[c, pl.ds(s * STR, STR)],
                    acc_sp.at[pl.ds(s * STR, STR)])
    pltpu.sync_copy(col_hbm.at[w], cidx_v)
    plsc.subcore_barrier()

    def ii(ch, islot):
        pltpu.async_copy(row_hbm.at[w, ch], ridx_v.at[islot], isem)

    def wi():
        pltpu.make_async_copy(row_hbm.at[0, 0], ridx_v.at[0], isem).wait()

    def ig(ch, slot):
        pltpu.async_copy(y_hbm.at[cidx_v.at[pl.ds(ch * CH, CH)]],
                         buf_v.at[pl.ds(slot * CH, CH)], gsem)

    def wg(slot):
        pltpu.make_async_copy(y_hbm.at[pl.ds(0, CH)],
                              buf_v.at[pl.ds(slot * CH, CH)], gsem).wait()

    def isc(ch, slot, islot):
        pltpu.async_copy(buf_v.at[pl.ds(slot * CH, CH)],
                         acc_sp.at[ridx_v.at[islot]], ssem, add=True)

    def wsc():
        pltpu.make_async_copy(y_hbm.at[pl.ds(0, CH)],
                              buf_v.at[pl.ds(0, CH)], ssem).wait()

    _pipelined_loop(ii, wi, ig, wg, isc, wsc)
    plsc.subcore_barrier()
    pltpu.sync_copy(acc_sp.at[pl.ds(s * STR, STR)],
                    out_hbm.at[c, pl.ds(s * STR, STR)])


def _rdeg_body(hist_ref, out_ref):
    out_ref[...] = lax.rsqrt(1.0 + hist_ref[0, :] + hist_ref[1, :])


def _rdeg_tc(hist2):
    return pl.pallas_call(
        _rdeg_body,
        out_shape=jax.ShapeDtypeStruct((NP,), jnp.float32),
    )(hist2)


def _in_conv_body(x_ref, wn_ref, bn_ref, wc_ref, bc_ref, rd_ref, o_ref):
    h0 = jax.nn.relu(
        jnp.dot(x_ref[...], wn_ref[...], preferred_element_type=jnp.float32)
        + bn_ref[...][None, :])
    y = (jnp.dot(h0, wc_ref[...], preferred_element_type=jnp.float32)
         + bc_ref[...][None, :]) * rd_ref[...]
    o_ref[...] = y


def _in_conv_tc(x, Wn, bn, Wc0, bc0, rdeg_col):
    blk = 1024
    return pl.pallas_call(
        _in_conv_body,
        grid=(NP // blk,),
        in_specs=[
            pl.BlockSpec((blk, H), lambda i: (i, 0)),
            pl.BlockSpec((H, H), lambda i: (0, 0)),
            pl.BlockSpec((H,), lambda i: (0,)),
            pl.BlockSpec((H, H), lambda i: (0, 0)),
            pl.BlockSpec((H,), lambda i: (0,)),
            pl.BlockSpec((blk, 1), lambda i: (i, 0)),
        ],
        out_specs=pl.BlockSpec((blk, H), lambda i: (i, 0)),
        out_shape=jax.ShapeDtypeStruct((NP, H), jnp.float32),
    )(x, Wn, bn, Wc0, bc0, rdeg_col)


def _msg_body(ea_ref, we_ref, be_ref, cn_ref, o_ref):
    m = jax.nn.relu(
        jnp.dot(ea_ref[...], we_ref[...], preferred_element_type=jnp.float32)
        + be_ref[...][None, :])
    o_ref[...] = m * cn_ref[...]


def _msg_tc(edge_attr, We, be, cnorm_col):
    blk = 2000
    return pl.pallas_call(
        _msg_body,
        grid=(E // blk,),
        in_specs=[
            pl.BlockSpec((blk, 16), lambda i: (i, 0)),
            pl.BlockSpec((16, H), lambda i: (0, 0)),
            pl.BlockSpec((H,), lambda i: (0,)),
            pl.BlockSpec((blk, 1), lambda i: (i, 0)),
        ],
        out_specs=pl.BlockSpec((blk, H), lambda i: (i, 0)),
        out_shape=jax.ShapeDtypeStruct((E, H), jnp.float32),
    )(edge_attr, We, be, cnorm_col)


def _combine_body(p_ref, rd_ref, w_ref, b_ref, o_ref):
    z = (p_ref[0] + p_ref[1]) * rd_ref[...]
    u = jax.nn.relu(z) + z
    o_ref[...] = (jnp.dot(u, w_ref[...], preferred_element_type=jnp.float32)
                  + b_ref[...][None, :]) * rd_ref[...]


def _combine_matmul_tc(p, rdeg_col, W, b):
    blk = 1024
    return pl.pallas_call(
        _combine_body,
        grid=(NP // blk,),
        in_specs=[
            pl.BlockSpec((NC, blk, H), lambda i: (0, i, 0)),
            pl.BlockSpec((blk, 1), lambda i: (i, 0)),
            pl.BlockSpec((H, H), lambda i: (0, 0)),
            pl.BlockSpec((H,), lambda i: (0,)),
        ],
        out_specs=pl.BlockSpec((blk, H), lambda i: (i, 0)),
        out_shape=jax.ShapeDtypeStruct((NP, H), jnp.float32),
    )(p, rdeg_col, W, b)


def _tail_body(p_ref, rd_ref, bt_ref, w1_ref, b1_ref, wf0_ref, bf0_ref,
               wf1_ref, bf1_ref, wo_ref, bo_ref, o_ref, pool_acc, cnt_acc):
    i = pl.program_id(0)
    blk = p_ref.shape[1]

    @pl.when(i == 0)
    def _():
        pool_acc[...] = jnp.zeros_like(pool_acc)
        cnt_acc[...] = jnp.zeros_like(cnt_acc)

    z = 2.0 * (p_ref[0] + p_ref[1]) * rd_ref[...]
    gids = lax.broadcasted_iota(jnp.int32, (1, G), 1)
    oh = (bt_ref[...] == gids).astype(jnp.float32)
    pool_acc[...] += lax.dot_general(
        oh, z, (((0,), (0,)), ((), ())), preferred_element_type=jnp.float32)
    cnt_acc[...] += lax.dot_general(
        oh, jnp.ones((blk, 1), jnp.float32), (((0,), (0,)), ((), ())),
        preferred_element_type=jnp.float32)

    @pl.when(i == pl.num_programs(0) - 1)
    def _():
        pooled = pool_acc[...] / jnp.maximum(cnt_acc[...], 1.0)
        hf = jax.nn.relu(
            jnp.dot(pooled, w1_ref[...], preferred_element_type=jnp.float32)
            + b1_ref[...][None, :])
        hf = jax.nn.relu(
            jnp.dot(hf, wf0_ref[...], preferred_element_type=jnp.float32)
            + bf0_ref[...][None, :])
        hf = jax.nn.relu(
            jnp.dot(hf, wf1_ref[...], preferred_element_type=jnp.float32)
            + bf1_ref[...][None, :])
        o_ref[...] = (jnp.dot(hf, wo_ref[...],
                              preferred_element_type=jnp.float32)
                      + bo_ref[...][None, :])


def _tail_tc(p, rdeg_col, batch_col, W1, b1, Wf0, bf0, Wf1, bf1, Wo, bo):
    blk = 1024
    FH = W1.shape[1]
    return pl.pallas_call(
        _tail_body,
        grid=(NP // blk,),
        in_specs=[
            pl.BlockSpec((NC, blk, H), lambda i: (0, i, 0)),
            pl.BlockSpec((blk, 1), lambda i: (i, 0)),
            pl.BlockSpec((blk, 1), lambda i: (i, 0)),
            pl.BlockSpec((H, FH), lambda i: (0, 0)),
            pl.BlockSpec((FH,), lambda i: (0,)),
            pl.BlockSpec((FH, FH), lambda i: (0, 0)),
            pl.BlockSpec((FH,), lambda i: (0,)),
            pl.BlockSpec((FH, FH), lambda i: (0, 0)),
            pl.BlockSpec((FH,), lambda i: (0,)),
            pl.BlockSpec((FH, 1), lambda i: (0, 0)),
            pl.BlockSpec((1,), lambda i: (0,)),
        ],
        out_specs=pl.BlockSpec((G, 1), lambda i: (0, 0)),
        out_shape=jax.ShapeDtypeStruct((G, 1), jnp.float32),
        scratch_shapes=[
            pltpu.VMEM((G, H), jnp.float32),
            pltpu.VMEM((G, 1), jnp.float32),
        ],
    )(p, rdeg_col, batch_col, W1, b1, Wf0, bf0, Wf1, bf1, Wo, bo)


def kernel(x, edge_index, edge_attr, batch, parity_atoms, add_feature,
           Wn, bn, We, be, Wc0, bc0, Wc1, bc1, Wc2, bc2,
           W1, b1, Wf0, bf0, Wf1, bf1, Wo, bo):
    row = edge_index[0]
    col = edge_index[1]
    row3 = row.reshape(NW, NCHUNK, CH)
    col3 = col.reshape(NW, NCHUNK, CH)
    col2 = col.reshape(NW, EW)

    hist2 = _hist_sc(row3)
    rdeg = _rdeg_tc(hist2)
    cnorm = _cnorm_sc(rdeg, col3)
    rdeg_col = rdeg.reshape(NP, 1)
    cnorm_col = cnorm.reshape(E, 1)

    y = _in_conv_tc(x, Wn, bn, Wc0, bc0, rdeg_col)
    msg = _msg_tc(edge_attr, We, be, cnorm_col)
    ebar, p = _edge_spmm_sc(msg, y, col2, row3)

    y = _combine_matmul_tc(p, rdeg_col, Wc1, bc1)
    p = _spmm_sc(y, col2, row3, ebar)
    y = _combine_matmul_tc(p, rdeg_col, Wc2, bc2)
    p = _spmm_sc(y, col2, row3, ebar)

    batch_col = jnp.concatenate(
        [batch, jnp.full((NP - N,), G, jnp.int32)]).reshape(NP, 1)
    return _tail_tc(p, rdeg_col, batch_col, W1, b1, Wf0, bf0, Wf1, bf1, Wo, bo)
